# Initial kernel scaffold; baseline (speedup 1.0000x reference)
import jax
import jax.numpy as jnp
from jax.experimental import pallas as pl


def kernel(x, edge_index, edge_weight, Wxz0, Wxz1, bxz, Whz0, Whz1, bhz, Wxr0, Wxr1, bxr, Whr0, Whr1, bhr, Wxh0, Wxh1, bxh, Whh0, Whh1, bhh, Wlin, blin):
    raise NotImplementedError("write your pallas kernel here")


# speedup vs baseline: 10.3712x; 10.3712x over previous
import functools

import jax
import jax.numpy as jnp
from jax import lax
from jax.experimental import pallas as pl
from jax.experimental.pallas import tpu as pltpu
from jax.experimental.pallas import tpu_sc as plsc

_NW = 32
_B = 128


def _deg_kernel(n_pad, kb):
    st = n_pad // 16
    mesh = plsc.VectorSubcoreMesh(core_axis_name="c", subcore_axis_name="s")

    @functools.partial(
        pl.kernel, mesh=mesh,
        compiler_params=pltpu.CompilerParams(use_tc_tiling_on_sc=False),
        out_type=jax.ShapeDtypeStruct((2, n_pad), jnp.float32),
        scratch_types=[
            pltpu.VMEM((_B,), jnp.int32),
            pltpu.VMEM((_B,), jnp.int32),
            pltpu.VMEM((_B,), jnp.float32),
            pltpu.VMEM((_B,), jnp.float32),
            pltpu.VMEM((st,), jnp.float32),
            pltpu.VMEM_SHARED((n_pad,), jnp.float32),
        ],
    )
    def k(row_hbm, col_hbm, w_hbm, out_hbm, row_v, col_v, w_v, wz_v, z_v, deg_sh):
        c = lax.axis_index("c")
        s = lax.axis_index("s")
        wid = c * 16 + s

        def zb(i, carry):
            z_v[pl.ds(i * 16, 16)] = jnp.zeros((16,), jnp.float32)
            return carry
        lax.fori_loop(0, st // 16, zb, 0)
        pltpu.sync_copy(z_v, deg_sh.at[pl.ds(s * st, st)])
        plsc.subcore_barrier()

        def blk(j, carry):
            pltpu.sync_copy(row_hbm.at[wid, j], row_v)
            pltpu.sync_copy(col_hbm.at[wid, j], col_v)
            pltpu.sync_copy(w_hbm.at[wid, j], w_v)
            for i in range(_B // 16):
                sl = pl.ds(i * 16, 16)
                wz_v[sl] = jnp.where(row_v[sl] == col_v[sl], 0.0, w_v[sl])
            pltpu.sync_copy(wz_v, deg_sh.at[row_v], add=True)
            return carry
        lax.fori_loop(0, kb, blk, 0)
        plsc.subcore_barrier()
        pltpu.sync_copy(deg_sh.at[pl.ds(s * st, st)],
                        out_hbm.at[c, pl.ds(s * st, st)])

    return k


def _edge_kernel(n_rows, n_pad, kb):
    st = n_pad // 16
    ncp = st // _B
    mesh = plsc.VectorSubcoreMesh(core_axis_name="c", subcore_axis_name="s")

    @functools.partial(
        pl.kernel, mesh=mesh,
        compiler_params=pltpu.CompilerParams(use_tc_tiling_on_sc=False),
        out_type=jax.ShapeDtypeStruct((2, n_pad, 64), jnp.float32),
        scratch_types=[
            pltpu.VMEM((_B,), jnp.int32),
            pltpu.VMEM((_B,), jnp.int32),
            pltpu.VMEM((_B,), jnp.float32),
            pltpu.VMEM((_B,), jnp.float32),
            pltpu.VMEM((_B, 64), jnp.float32),
            pltpu.VMEM((_B, 64), jnp.float32),
            pltpu.VMEM_SHARED((n_pad, 64), jnp.float32),
            pltpu.SemaphoreType.DMA,
        ],
    )
    def k(row_hbm, col_hbm, w_hbm, p_hbm, out_hbm,
          row_v, col_v, w_v, wn_v, rows_v, zero_v, s_sh, sem):
        c = lax.axis_index("c")
        s = lax.axis_index("s")
        wid = c * 16 + s

        def zb(i, carry):
            for jj in range(4):
                zero_v[i, pl.ds(jj * 16, 16)] = jnp.zeros((16,), jnp.float32)
            return carry
        lax.fori_loop(0, _B, zb, 0)
        for kcp in range(ncp):
            pltpu.sync_copy(zero_v, s_sh.at[pl.ds(s * st + kcp * _B, _B)])
        plsc.subcore_barrier()

        def blk(j, carry):
            pltpu.sync_copy(row_hbm.at[wid, j], row_v)
            pltpu.sync_copy(col_hbm.at[wid, j], col_v)
            pltpu.sync_copy(w_hbm.at[wid, j], w_v)
            for i in range(_B // 16):
                sl = pl.ds(i * 16, 16)
                wn_v[sl] = jnp.where(row_v[sl] == col_v[sl], 0.0, -w_v[sl])
            pltpu.async_copy(p_hbm.at[row_v], rows_v, sem).wait()

            def scale(g, carry2):
                chunk = wn_v[pl.ds(g * 16, 16)]
                base = g * 16
                for k in range(16):
                    f = chunk[k]
                    for jj in range(4):
                        sl2 = pl.ds(jj * 16, 16)
                        rows_v[base + k, sl2] = rows_v[base + k, sl2] * f
                return carry2
            lax.fori_loop(0, _B // 16, scale, 0)
            pltpu.sync_copy(rows_v, s_sh.at[col_v], add=True)
            return carry
        lax.fori_loop(0, kb, blk, 0)
        plsc.subcore_barrier()
        pltpu.sync_copy(s_sh.at[pl.ds(s * st, st)],
                        out_hbm.at[c, pl.ds(s * st, st)])

    return k


def _dis_block(degT_ref):
    deg = degT_ref[:, 0:1] + degT_ref[:, 1:2]
    safe = jnp.where(deg > 0, deg, 1.0)
    return jnp.where(deg > 0, lax.rsqrt(safe), 0.0)


def _tc_dense(xs, wc0, wc1, degT, bn):
    n, f = xs.shape

    def body(xs_ref, w0_ref, w1_ref, degT_ref, d_ref, p_ref):
        dis = _dis_block(degT_ref)
        xv = xs_ref[...]
        d_ref[...] = jnp.dot(xv, w0_ref[...], preferred_element_type=jnp.float32)
        p_ref[...] = dis * jnp.dot(xv, w1_ref[...],
                                   preferred_element_type=jnp.float32)

    return pl.pallas_call(
        body,
        grid=(n // bn,),
        in_specs=[
            pl.BlockSpec((bn, f), lambda i: (i, 0)),
            pl.BlockSpec((f, 64), lambda i: (0, 0)),
            pl.BlockSpec((f, 64), lambda i: (0, 0)),
            pl.BlockSpec((bn, 2), lambda i: (i, 0)),
        ],
        out_specs=[
            pl.BlockSpec((bn, 64), lambda i: (i, 0)),
            pl.BlockSpec((bn, 64), lambda i: (i, 0)),
        ],
        out_shape=[
            jax.ShapeDtypeStruct((n, 64), jnp.float32),
            jax.ShapeDtypeStruct((n, 64), jnp.float32),
        ],
    )(xs, wc0, wc1, degT)


def _tc_gates(d, s2, degT, bz, bh, wlin, blin, bn):
    n = d.shape[0]
    hor = wlin.shape[1]

    def body(d_ref, s_ref, degT_ref, bz_ref, bh_ref, wl_ref, bl_ref, o_ref):
        dis = _dis_block(degT_ref)
        sv = (s_ref[0] + s_ref[1]) * dis
        a = d_ref[...] + sv
        z = jax.nn.sigmoid(a[:, :32] + bz_ref[...])
        ht = jnp.tanh(a[:, 32:] + bh_ref[...])
        h = jnp.maximum((1.0 - z) * ht, 0.0)
        o_ref[...] = jnp.dot(h, wl_ref[...],
                             preferred_element_type=jnp.float32) + bl_ref[...]

    return pl.pallas_call(
        body,
        grid=(n // bn,),
        in_specs=[
            pl.BlockSpec((bn, 64), lambda i: (i, 0)),
            pl.BlockSpec((2, bn, 64), lambda i: (0, i, 0)),
            pl.BlockSpec((bn, 2), lambda i: (i, 0)),
            pl.BlockSpec((1, 32), lambda i: (0, 0)),
            pl.BlockSpec((1, 32), lambda i: (0, 0)),
            pl.BlockSpec((32, hor), lambda i: (0, 0)),
            pl.BlockSpec((1, hor), lambda i: (0, 0)),
        ],
        out_specs=pl.BlockSpec((bn, hor), lambda i: (i, 0)),
        out_shape=jax.ShapeDtypeStruct((n, hor), jnp.float32),
    )(d, s2, degT, bz, bh, wlin, blin)


def kernel(x, edge_index, edge_weight, Wxz0, Wxz1, bxz, Whz0, Whz1, bhz,
           Wxr0, Wxr1, bxr, Whr0, Whr1, bhr, Wxh0, Wxh1, bxh,
           Whh0, Whh1, bhh, Wlin, blin):
    xs = jnp.squeeze(x, 1)
    n = xs.shape[0]
    e = edge_index.shape[1]

    kb = -(-e // (_NW * _B))
    ep = _NW * kb * _B
    n_pad = -(-n // 2048) * 2048

    row3 = jnp.pad(edge_index[0], (0, ep - e)).reshape(_NW, kb, _B)
    col3 = jnp.pad(edge_index[1], (0, ep - e)).reshape(_NW, kb, _B)
    w3 = jnp.pad(edge_weight, (0, ep - e)).reshape(_NW, kb, _B)

    wc0 = jnp.concatenate([Wxz0, Wxh0], axis=1)
    wc1 = jnp.concatenate([Wxz1, Wxh1], axis=1)
    bz = (bxz + bhz).reshape(1, -1)
    bh = (bxh + bhh).reshape(1, -1)
    blin2 = blin.reshape(1, -1)

    deg2 = _deg_kernel(n_pad, kb)(row3, col3, w3)
    degT = deg2.T[:n]

    bn = 400
    d, p = _tc_dense(xs, wc0, wc1, degT, bn)
    s2 = _edge_kernel(n, n_pad, kb)(row3, col3, w3, p)
    return _tc_gates(d, s2, degT, bz, bh, Wlin, blin2, bn)


# speedup vs baseline: 18.4413x; 1.7781x over previous
import functools

import jax
import jax.numpy as jnp
from jax import lax
from jax.experimental import pallas as pl
from jax.experimental.pallas import tpu as pltpu
from jax.experimental.pallas import tpu_sc as plsc

_NW = 32
_B = 128


def _deg_kernel(n_pad, kb):
    st = n_pad // 16
    mesh = plsc.VectorSubcoreMesh(core_axis_name="c", subcore_axis_name="s")

    @functools.partial(
        pl.kernel, mesh=mesh,
        compiler_params=pltpu.CompilerParams(use_tc_tiling_on_sc=False),
        out_type=jax.ShapeDtypeStruct((2, n_pad), jnp.float32),
        scratch_types=[
            pltpu.VMEM((kb, _B), jnp.int32),
            pltpu.VMEM((kb, _B), jnp.int32),
            pltpu.VMEM((kb, _B), jnp.float32),
            pltpu.VMEM((kb, _B), jnp.float32),
            pltpu.VMEM((st,), jnp.float32),
            pltpu.VMEM_SHARED((n_pad,), jnp.float32),
            pltpu.SemaphoreType.DMA,
        ],
    )
    def k(row_hbm, col_hbm, w_hbm, out_hbm, row2, col2, w2, wz2, z_v, deg_sh,
          sem):
        c = lax.axis_index("c")
        s = lax.axis_index("s")
        wid = c * 16 + s
        pltpu.sync_copy(row_hbm.at[wid], row2)
        pltpu.sync_copy(col_hbm.at[wid], col2)
        pltpu.sync_copy(w_hbm.at[wid], w2)

        def zb(i, carry):
            z_v[pl.ds(i * 16, 16)] = jnp.zeros((16,), jnp.float32)
            return carry
        lax.fori_loop(0, st // 16, zb, 0)
        pltpu.sync_copy(z_v, deg_sh.at[pl.ds(s * st, st)])
        plsc.subcore_barrier()

        def blk(j, carry):
            for i in range(_B // 16):
                sl = pl.ds(i * 16, 16)
                wz2[j, sl] = jnp.where(row2[j, sl] == col2[j, sl], 0.0,
                                       w2[j, sl])
            pltpu.async_copy(wz2.at[j], deg_sh.at[row2.at[j]], sem, add=True)

            @pl.when(j >= 8)
            def _():
                pltpu.make_async_copy(wz2.at[0], deg_sh.at[row2.at[0]],
                                      sem).wait()
            return carry
        lax.fori_loop(0, kb, blk, 0)

        def drain(j, carry):
            pltpu.make_async_copy(wz2.at[0], deg_sh.at[row2.at[0]],
                                  sem).wait()
            return carry
        lax.fori_loop(0, min(kb, 8), drain, 0)
        plsc.subcore_barrier()
        pltpu.sync_copy(deg_sh.at[pl.ds(s * st, st)],
                        out_hbm.at[c, pl.ds(s * st, st)])

    return k


def _edge_kernel(n_rows, n_pad, kb):
    st = n_pad // 16
    ncp = st // _B
    mesh = plsc.VectorSubcoreMesh(core_axis_name="c", subcore_axis_name="s")

    @functools.partial(
        pl.kernel, mesh=mesh,
        compiler_params=pltpu.CompilerParams(use_tc_tiling_on_sc=False),
        out_type=jax.ShapeDtypeStruct((2, n_pad, 64), jnp.float32),
        scratch_types=[
            pltpu.VMEM((kb, _B), jnp.int32),
            pltpu.VMEM((kb, _B), jnp.int32),
            pltpu.VMEM((kb, _B), jnp.float32),
            pltpu.VMEM((_B,), jnp.float32),
            pltpu.VMEM((_B, 64), jnp.float32),
            pltpu.VMEM((_B, 64), jnp.float32),
            pltpu.VMEM((_B, 64), jnp.float32),
            pltpu.VMEM_SHARED((n_pad, 64), jnp.float32),
            pltpu.SemaphoreType.DMA,
            pltpu.SemaphoreType.DMA,
        ],
    )
    def k(row_hbm, col_hbm, w_hbm, p_hbm, out_hbm,
          row2, col2, w2, wn_v, rows_a, rows_b, zero_v, s_sh, sem_a, sem_b):
        c = lax.axis_index("c")
        s = lax.axis_index("s")
        wid = c * 16 + s
        pltpu.sync_copy(row_hbm.at[wid], row2)
        pltpu.sync_copy(col_hbm.at[wid], col2)
        pltpu.sync_copy(w_hbm.at[wid], w2)

        def zb(i, carry):
            for jj in range(4):
                zero_v[i, pl.ds(jj * 16, 16)] = jnp.zeros((16,), jnp.float32)
            return carry
        lax.fori_loop(0, _B, zb, 0)
        for kcp in range(ncp):
            pltpu.sync_copy(zero_v, s_sh.at[pl.ds(s * st + kcp * _B, _B)])
        plsc.subcore_barrier()

        rbufs = (rows_a, rows_b)
        sems = (sem_a, sem_b)
        pltpu.async_copy(p_hbm.at[row2.at[0]], rows_a, sem_a)

        def rnd(g, carry):
            for b in range(2):
                j = g * 2 + b
                nb = 1 - b

                @pl.when(j + 1 < kb)
                def _():
                    pltpu.async_copy(p_hbm.at[row2.at[j + 1]], rbufs[nb],
                                     sems[nb])
                pltpu.make_async_copy(p_hbm.at[row2.at[j]], rbufs[b],
                                      sems[b]).wait()
                for i in range(_B // 16):
                    sl = pl.ds(i * 16, 16)
                    wn_v[sl] = jnp.where(row2[j, sl] == col2[j, sl], 0.0,
                                         -w2[j, sl])

                def scale(g2, carry2):
                    chunk = wn_v[pl.ds(g2 * 16, 16)]
                    base = g2 * 16
                    for k2 in range(16):
                        f = chunk[k2]
                        for jj in range(4):
                            sl2 = pl.ds(jj * 16, 16)
                            rbufs[b][base + k2, sl2] = (
                                rbufs[b][base + k2, sl2] * f)
                    return carry2
                lax.fori_loop(0, _B // 16, scale, 0)
                pltpu.sync_copy(rbufs[b], s_sh.at[col2.at[j]], add=True)
            return carry
        lax.fori_loop(0, kb // 2, rnd, 0)
        plsc.subcore_barrier()
        pltpu.sync_copy(s_sh.at[pl.ds(s * st, st)],
                        out_hbm.at[c, pl.ds(s * st, st)])

    return k


def _dis_block(degT_ref):
    deg = degT_ref[:, 0:1] + degT_ref[:, 1:2]
    safe = jnp.where(deg > 0, deg, 1.0)
    return jnp.where(deg > 0, lax.rsqrt(safe), 0.0)


def _tc_dense(xs, wc0, wc1, degT, bn):
    n, f = xs.shape

    def body(xs_ref, w0_ref, w1_ref, degT_ref, d_ref, p_ref):
        dis = _dis_block(degT_ref)
        xv = xs_ref[...]
        d_ref[...] = jnp.dot(xv, w0_ref[...], preferred_element_type=jnp.float32)
        p_ref[...] = dis * jnp.dot(xv, w1_ref[...],
                                   preferred_element_type=jnp.float32)

    return pl.pallas_call(
        body,
        grid=(n // bn,),
        in_specs=[
            pl.BlockSpec((bn, f), lambda i: (i, 0)),
            pl.BlockSpec((f, 64), lambda i: (0, 0)),
            pl.BlockSpec((f, 64), lambda i: (0, 0)),
            pl.BlockSpec((bn, 2), lambda i: (i, 0)),
        ],
        out_specs=[
            pl.BlockSpec((bn, 64), lambda i: (i, 0)),
            pl.BlockSpec((bn, 64), lambda i: (i, 0)),
        ],
        out_shape=[
            jax.ShapeDtypeStruct((n, 64), jnp.float32),
            jax.ShapeDtypeStruct((n, 64), jnp.float32),
        ],
    )(xs, wc0, wc1, degT)


def _tc_gates(d, s2, degT, bz, bh, wlin, blin, bn):
    n = d.shape[0]
    hor = wlin.shape[1]

    def body(d_ref, s_ref, degT_ref, bz_ref, bh_ref, wl_ref, bl_ref, o_ref):
        dis = _dis_block(degT_ref)
        sv = (s_ref[0] + s_ref[1]) * dis
        a = d_ref[...] + sv
        z = jax.nn.sigmoid(a[:, :32] + bz_ref[...])
        ht = jnp.tanh(a[:, 32:] + bh_ref[...])
        h = jnp.maximum((1.0 - z) * ht, 0.0)
        o_ref[...] = jnp.dot(h, wl_ref[...],
                             preferred_element_type=jnp.float32) + bl_ref[...]

    return pl.pallas_call(
        body,
        grid=(n // bn,),
        in_specs=[
            pl.BlockSpec((bn, 64), lambda i: (i, 0)),
            pl.BlockSpec((2, bn, 64), lambda i: (0, i, 0)),
            pl.BlockSpec((bn, 2), lambda i: (i, 0)),
            pl.BlockSpec((1, 32), lambda i: (0, 0)),
            pl.BlockSpec((1, 32), lambda i: (0, 0)),
            pl.BlockSpec((32, hor), lambda i: (0, 0)),
            pl.BlockSpec((1, hor), lambda i: (0, 0)),
        ],
        out_specs=pl.BlockSpec((bn, hor), lambda i: (i, 0)),
        out_shape=jax.ShapeDtypeStruct((n, hor), jnp.float32),
    )(d, s2, degT, bz, bh, wlin, blin)


def kernel(x, edge_index, edge_weight, Wxz0, Wxz1, bxz, Whz0, Whz1, bhz,
           Wxr0, Wxr1, bxr, Whr0, Whr1, bhr, Wxh0, Wxh1, bxh,
           Whh0, Whh1, bhh, Wlin, blin):
    xs = jnp.squeeze(x, 1)
    n = xs.shape[0]
    e = edge_index.shape[1]

    kb = -(-e // (_NW * _B))
    kb += kb % 2
    ep = _NW * kb * _B
    n_pad = -(-n // 2048) * 2048

    row3 = jnp.pad(edge_index[0], (0, ep - e)).reshape(_NW, kb, _B)
    col3 = jnp.pad(edge_index[1], (0, ep - e)).reshape(_NW, kb, _B)
    w3 = jnp.pad(edge_weight, (0, ep - e)).reshape(_NW, kb, _B)

    wc0 = jnp.concatenate([Wxz0, Wxh0], axis=1)
    wc1 = jnp.concatenate([Wxz1, Wxh1], axis=1)
    bz = (bxz + bhz).reshape(1, -1)
    bh = (bxh + bhh).reshape(1, -1)
    blin2 = blin.reshape(1, -1)

    deg2 = _deg_kernel(n_pad, kb)(row3, col3, w3)
    degT = deg2.T[:n]

    bn = 400
    d, p = _tc_dense(xs, wc0, wc1, degT, bn)
    s2 = _edge_kernel(n, n_pad, kb)(row3, col3, w3, p)
    return _tc_gates(d, s2, degT, bz, bh, Wlin, blin2, bn)


# speedup vs baseline: 18.6269x; 1.0101x over previous
import functools

import jax
import jax.numpy as jnp
from jax import lax
from jax.experimental import pallas as pl
from jax.experimental.pallas import tpu as pltpu
from jax.experimental.pallas import tpu_sc as plsc

_NW = 32
_B = 128


def _deg_kernel(n_pad, kb):
    st = n_pad // 16
    mesh = plsc.VectorSubcoreMesh(core_axis_name="c", subcore_axis_name="s")

    @functools.partial(
        pl.kernel, mesh=mesh,
        compiler_params=pltpu.CompilerParams(use_tc_tiling_on_sc=False),
        out_type=jax.ShapeDtypeStruct((2, n_pad), jnp.float32),
        scratch_types=[
            pltpu.VMEM((kb, _B), jnp.int32),
            pltpu.VMEM((kb, _B), jnp.int32),
            pltpu.VMEM((kb, _B), jnp.float32),
            pltpu.VMEM((kb, _B), jnp.float32),
            pltpu.VMEM((st,), jnp.float32),
            pltpu.VMEM_SHARED((n_pad,), jnp.float32),
            pltpu.SemaphoreType.DMA,
        ],
    )
    def k(row_hbm, col_hbm, w_hbm, out_hbm, row2, col2, w2, wz2, z_v, deg_sh,
          sem):
        c = lax.axis_index("c")
        s = lax.axis_index("s")
        wid = c * 16 + s
        pltpu.sync_copy(row_hbm.at[wid], row2)
        pltpu.sync_copy(col_hbm.at[wid], col2)
        pltpu.sync_copy(w_hbm.at[wid], w2)

        def zb(i, carry):
            z_v[pl.ds(i * 16, 16)] = jnp.zeros((16,), jnp.float32)
            return carry
        lax.fori_loop(0, st // 16, zb, 0)
        pltpu.sync_copy(z_v, deg_sh.at[pl.ds(s * st, st)])
        plsc.subcore_barrier()

        def blk(j, carry):
            for i in range(_B // 16):
                sl = pl.ds(i * 16, 16)
                wz2[j, sl] = jnp.where(row2[j, sl] == col2[j, sl], 0.0,
                                       w2[j, sl])
            pltpu.async_copy(wz2.at[j], deg_sh.at[row2.at[j]], sem, add=True)

            @pl.when(j >= 8)
            def _():
                pltpu.make_async_copy(wz2.at[0], deg_sh.at[row2.at[0]],
                                      sem).wait()
            return carry
        lax.fori_loop(0, kb, blk, 0)

        def drain(j, carry):
            pltpu.make_async_copy(wz2.at[0], deg_sh.at[row2.at[0]],
                                  sem).wait()
            return carry
        lax.fori_loop(0, min(kb, 8), drain, 0)
        plsc.subcore_barrier()
        pltpu.sync_copy(deg_sh.at[pl.ds(s * st, st)],
                        out_hbm.at[c, pl.ds(s * st, st)])

    return k


def _edge_kernel(n_rows, n_pad, kb):
    st = n_pad // 16
    ncp = st // _B
    mesh = plsc.VectorSubcoreMesh(core_axis_name="c", subcore_axis_name="s")

    @functools.partial(
        pl.kernel, mesh=mesh,
        compiler_params=pltpu.CompilerParams(use_tc_tiling_on_sc=False),
        out_type=jax.ShapeDtypeStruct((2, n_pad, 64), jnp.float32),
        scratch_types=[
            pltpu.VMEM((kb, _B), jnp.int32),
            pltpu.VMEM((kb, _B), jnp.int32),
            pltpu.VMEM((kb, _B), jnp.float32),
            pltpu.VMEM((_B,), jnp.float32),
            pltpu.VMEM((_B, 64), jnp.float32),
            pltpu.VMEM((_B, 64), jnp.float32),
            pltpu.VMEM((_B, 64), jnp.float32),
            pltpu.VMEM((_B, 64), jnp.float32),
            pltpu.VMEM((_B, 64), jnp.float32),
            pltpu.VMEM_SHARED((n_pad, 64), jnp.float32),
            pltpu.SemaphoreType.DMA,
            pltpu.SemaphoreType.DMA,
            pltpu.SemaphoreType.DMA,
            pltpu.SemaphoreType.DMA,
            pltpu.SemaphoreType.DMA,
            pltpu.SemaphoreType.DMA,
            pltpu.SemaphoreType.DMA,
            pltpu.SemaphoreType.DMA,
        ],
    )
    def k(row_hbm, col_hbm, w_hbm, p_hbm, out_hbm,
          row2, col2, w2, wn_v, rows_0, rows_1, rows_2, rows_3, zero_v, s_sh,
          sg0, sg1, sg2, sg3, ss0, ss1, ss2, ss3):
        c = lax.axis_index("c")
        s = lax.axis_index("s")
        wid = c * 16 + s
        pltpu.sync_copy(row_hbm.at[wid], row2)
        pltpu.sync_copy(col_hbm.at[wid], col2)
        pltpu.sync_copy(w_hbm.at[wid], w2)

        def zb(i, carry):
            for jj in range(4):
                zero_v[i, pl.ds(jj * 16, 16)] = jnp.zeros((16,), jnp.float32)
            return carry
        lax.fori_loop(0, _B, zb, 0)
        for kcp in range(ncp):
            pltpu.sync_copy(zero_v, s_sh.at[pl.ds(s * st + kcp * _B, _B)])
        plsc.subcore_barrier()

        rbufs = (rows_0, rows_1, rows_2, rows_3)
        sgs = (sg0, sg1, sg2, sg3)
        sss = (ss0, ss1, ss2, ss3)
        pltpu.async_copy(p_hbm.at[row2.at[0]], rbufs[0], sgs[0])
        pltpu.async_copy(p_hbm.at[row2.at[1]], rbufs[1], sgs[1])

        def rnd(g, carry):
            for b in range(4):
                j = g * 4 + b
                pltpu.make_async_copy(p_hbm.at[row2.at[j]], rbufs[b],
                                      sgs[b]).wait()
                for i in range(_B // 16):
                    sl = pl.ds(i * 16, 16)
                    wn_v[sl] = jnp.where(row2[j, sl] == col2[j, sl], 0.0,
                                         -w2[j, sl])

                def scale(g2, carry2):
                    chunk = wn_v[pl.ds(g2 * 16, 16)]
                    base = g2 * 16
                    for k2 in range(16):
                        f = chunk[k2]
                        for jj in range(4):
                            sl2 = pl.ds(jj * 16, 16)
                            rbufs[b][base + k2, sl2] = (
                                rbufs[b][base + k2, sl2] * f)
                    return carry2
                lax.fori_loop(0, _B // 16, scale, 0)

                nb = (b + 2) % 4

                @pl.when(j >= 2)
                def _():
                    pltpu.make_async_copy(rbufs[nb], s_sh.at[col2.at[j - 2]],
                                          sss[nb]).wait()
                pltpu.async_copy(rbufs[b], s_sh.at[col2.at[j]], sss[b],
                                 add=True)

                @pl.when(j + 2 < kb)
                def _():
                    pltpu.async_copy(p_hbm.at[row2.at[j + 2]], rbufs[nb],
                                     sgs[nb])
            return carry
        lax.fori_loop(0, kb // 4, rnd, 0)
        pltpu.make_async_copy(rbufs[2], s_sh.at[col2.at[kb - 2]],
                              sss[2]).wait()
        pltpu.make_async_copy(rbufs[3], s_sh.at[col2.at[kb - 1]],
                              sss[3]).wait()
        plsc.subcore_barrier()
        pltpu.sync_copy(s_sh.at[pl.ds(s * st, st)],
                        out_hbm.at[c, pl.ds(s * st, st)])

    return k


def _dis_block(degT_ref):
    deg = degT_ref[:, 0:1] + degT_ref[:, 1:2]
    safe = jnp.where(deg > 0, deg, 1.0)
    return jnp.where(deg > 0, lax.rsqrt(safe), 0.0)


def _tc_dense(xs, wc0, wc1, degT, bn):
    n, f = xs.shape

    def body(xs_ref, w0_ref, w1_ref, degT_ref, d_ref, p_ref):
        dis = _dis_block(degT_ref)
        xv = xs_ref[...]
        d_ref[...] = jnp.dot(xv, w0_ref[...], preferred_element_type=jnp.float32)
        p_ref[...] = dis * jnp.dot(xv, w1_ref[...],
                                   preferred_element_type=jnp.float32)

    return pl.pallas_call(
        body,
        grid=(n // bn,),
        in_specs=[
            pl.BlockSpec((bn, f), lambda i: (i, 0)),
            pl.BlockSpec((f, 64), lambda i: (0, 0)),
            pl.BlockSpec((f, 64), lambda i: (0, 0)),
            pl.BlockSpec((bn, 2), lambda i: (i, 0)),
        ],
        out_specs=[
            pl.BlockSpec((bn, 64), lambda i: (i, 0)),
            pl.BlockSpec((bn, 64), lambda i: (i, 0)),
        ],
        out_shape=[
            jax.ShapeDtypeStruct((n, 64), jnp.float32),
            jax.ShapeDtypeStruct((n, 64), jnp.float32),
        ],
    )(xs, wc0, wc1, degT)


def _tc_gates(d, s2, degT, bz, bh, wlin, blin, bn):
    n = d.shape[0]
    hor = wlin.shape[1]

    def body(d_ref, s_ref, degT_ref, bz_ref, bh_ref, wl_ref, bl_ref, o_ref):
        dis = _dis_block(degT_ref)
        sv = (s_ref[0] + s_ref[1]) * dis
        a = d_ref[...] + sv
        z = jax.nn.sigmoid(a[:, :32] + bz_ref[...])
        ht = jnp.tanh(a[:, 32:] + bh_ref[...])
        h = jnp.maximum((1.0 - z) * ht, 0.0)
        o_ref[...] = jnp.dot(h, wl_ref[...],
                             preferred_element_type=jnp.float32) + bl_ref[...]

    return pl.pallas_call(
        body,
        grid=(n // bn,),
        in_specs=[
            pl.BlockSpec((bn, 64), lambda i: (i, 0)),
            pl.BlockSpec((2, bn, 64), lambda i: (0, i, 0)),
            pl.BlockSpec((bn, 2), lambda i: (i, 0)),
            pl.BlockSpec((1, 32), lambda i: (0, 0)),
            pl.BlockSpec((1, 32), lambda i: (0, 0)),
            pl.BlockSpec((32, hor), lambda i: (0, 0)),
            pl.BlockSpec((1, hor), lambda i: (0, 0)),
        ],
        out_specs=pl.BlockSpec((bn, hor), lambda i: (i, 0)),
        out_shape=jax.ShapeDtypeStruct((n, hor), jnp.float32),
    )(d, s2, degT, bz, bh, wlin, blin)


def kernel(x, edge_index, edge_weight, Wxz0, Wxz1, bxz, Whz0, Whz1, bhz,
           Wxr0, Wxr1, bxr, Whr0, Whr1, bhr, Wxh0, Wxh1, bxh,
           Whh0, Whh1, bhh, Wlin, blin):
    xs = jnp.squeeze(x, 1)
    n = xs.shape[0]
    e = edge_index.shape[1]

    kb = -(-e // (_NW * _B))
    kb += (-kb) % 4
    ep = _NW * kb * _B
    n_pad = -(-n // 2048) * 2048

    row3 = jnp.pad(edge_index[0], (0, ep - e)).reshape(_NW, kb, _B)
    col3 = jnp.pad(edge_index[1], (0, ep - e)).reshape(_NW, kb, _B)
    w3 = jnp.pad(edge_weight, (0, ep - e)).reshape(_NW, kb, _B)

    wc0 = jnp.concatenate([Wxz0, Wxh0], axis=1)
    wc1 = jnp.concatenate([Wxz1, Wxh1], axis=1)
    bz = (bxz + bhz).reshape(1, -1)
    bh = (bxh + bhh).reshape(1, -1)
    blin2 = blin.reshape(1, -1)

    deg2 = _deg_kernel(n_pad, kb)(row3, col3, w3)
    degT = deg2.T[:n]

    bn = 400
    d, p = _tc_dense(xs, wc0, wc1, degT, bn)
    s2 = _edge_kernel(n, n_pad, kb)(row3, col3, w3, p)
    return _tc_gates(d, s2, degT, bz, bh, Wlin, blin2, bn)
